# baseline (speedup 1.0000x reference)
import functools

import jax
import jax.numpy as jnp
from jax import lax
from jax.experimental import pallas as pl
from jax.experimental.pallas import tpu as pltpu
from jax.experimental.pallas import tpu_sc as plsc

F32 = jnp.float32
I32 = jnp.int32

NC = 2
NS = 16
CW = 128
G = 4


def _ceil_to(x, m):
    return ((x + m - 1) // m) * m


@functools.lru_cache(maxsize=None)
def _build_gather_deg(num_ent, n, d, n_chunks):
    rows_chunk = 80
    x_chunks = -(-n // rows_chunk)
    xg_iters = -(-x_chunks // (NC * NS))
    npad = _ceil_to(n + 1, 640)
    cpt = n_chunks // (NC * NS)

    mesh = plsc.VectorSubcoreMesh(core_axis_name="c", subcore_axis_name="s",
                                  num_cores=NC, num_subcores=NS)

    @functools.partial(
        pl.kernel,
        mesh=mesh,
        out_type=(
            jax.ShapeDtypeStruct((n, d), F32),
            jax.ShapeDtypeStruct((npad,), F32),
            jax.ShapeDtypeStruct((npad,), F32),
        ),
        scratch_types=[
            pltpu.VMEM((rows_chunk,), I32),
            pltpu.VMEM((rows_chunk, d), F32),
            pltpu.VMEM((cpt, CW), I32),
            pltpu.VMEM((CW,), F32),
            pltpu.VMEM_SHARED((npad,), F32),
            pltpu.SemaphoreType.DMA,
            pltpu.SemaphoreType.DMA,
        ],
    )
    def k(tab_hbm, ids_hbm, dst2d_hbm, zeros_hbm, ones_hbm,
          x_hbm, cnt0_hbm, cnt1_hbm,
          idbuf, rowbuf, dstbuf, onesbuf, cnt_sp, sem, ssem):
        c = lax.axis_index("c")
        s = lax.axis_index("s")
        w = s * NC + c

        pltpu.sync_copy(zeros_hbm, cnt_sp.at[pl.ds(s * 640, 640)])
        pltpu.sync_copy(ones_hbm, onesbuf)
        base_chunk = (c * NS + s) * cpt
        pltpu.sync_copy(dst2d_hbm.at[pl.ds(base_chunk, cpt)], dstbuf)
        plsc.subcore_barrier()

        def dg(j, carry):
            pltpu.async_copy(onesbuf, cnt_sp.at[dstbuf.at[j]], ssem, add=True)
            return carry
        lax.fori_loop(0, cpt, dg, 0)

        def xg(j, carry):
            cid = w + NC * NS * j
            @pl.when(cid < x_chunks)
            def _():
                base = cid * rows_chunk
                pltpu.sync_copy(ids_hbm.at[pl.ds(base, rows_chunk)], idbuf)
                pltpu.async_copy(tab_hbm.at[idbuf], rowbuf, sem).wait()
                pltpu.sync_copy(rowbuf, x_hbm.at[pl.ds(base, rows_chunk)])
            return carry
        lax.fori_loop(0, xg_iters, xg, 0)

        def dw(j, carry):
            pltpu.make_async_copy(onesbuf, cnt_sp.at[dstbuf.at[j]], ssem).wait()
            return carry
        lax.fori_loop(0, cpt, dw, 0)
        plsc.subcore_barrier()
        @pl.when(c == 0)
        def _():
            pltpu.sync_copy(cnt_sp.at[pl.ds(s * 640, 640)],
                            cnt0_hbm.at[pl.ds(s * 640, 640)])
        @pl.when(c == 1)
        def _():
            pltpu.sync_copy(cnt_sp.at[pl.ds(s * 640, 640)],
                            cnt1_hbm.at[pl.ds(s * 640, 640)])

    return k


@functools.lru_cache(maxsize=None)
def _build_message(n, h, n_chunks):
    assert n_chunks % (NS * 2) == 0
    cpt = n_chunks // NS
    hcpt = cpt // 2
    npad = _ceil_to(n + 1, 8)
    rpt = (n // NS) & ~7
    tail = n - NS * rpt
    zpad = npad - n + tail

    mesh = plsc.VectorSubcoreMesh(core_axis_name="c", subcore_axis_name="s",
                                  num_cores=NC, num_subcores=NS)

    @functools.partial(
        pl.kernel,
        mesh=mesh,
        out_type=(
            jax.ShapeDtypeStruct((n, h), F32),
            jax.ShapeDtypeStruct((n, h), F32),
        ),
        scratch_types=[
            pltpu.VMEM((hcpt, CW), I32),
            pltpu.VMEM((hcpt, CW), I32),
            pltpu.VMEM((2, CW, h), F32),
            pltpu.VMEM_SHARED((npad, h), F32),
            pltpu.SemaphoreType.DMA,
            pltpu.SemaphoreType.DMA,
        ],
    )
    def k(ga_hbm, gb_hbm, src2d_hbm, dst2d_hbm, zrows_hbm,
          acca_hbm, accb_hbm,
          sidx, didx, rows, acc_sp, gsem, ssem):
        c = lax.axis_index("c")
        s = lax.axis_index("s")
        r0 = s * rpt

        pltpu.sync_copy(zrows_hbm.at[pl.ds(0, rpt)], acc_sp.at[pl.ds(r0, rpt)])
        @pl.when(s == NS - 1)
        def _():
            pltpu.sync_copy(zrows_hbm.at[pl.ds(0, zpad)],
                            acc_sp.at[pl.ds(NS * rpt, zpad)])
        plsc.subcore_barrier()

        def run(g_hbm):
            for half in range(2):
                base_chunk = s * cpt + half * hcpt
                pltpu.sync_copy(src2d_hbm.at[pl.ds(base_chunk, hcpt)], sidx)
                pltpu.sync_copy(dst2d_hbm.at[pl.ds(base_chunk, hcpt)], didx)

                def body(j, carry):
                    b = lax.rem(j, 2)
                    @pl.when(j >= 2)
                    def _():
                        pltpu.make_async_copy(
                            rows.at[b], acc_sp.at[didx.at[j - 2]],
                            ssem).wait()
                    pltpu.sync_copy(g_hbm.at[sidx.at[j]], rows.at[b])
                    pltpu.async_copy(rows.at[b], acc_sp.at[didx.at[j]],
                                     ssem, add=True)
                    return carry
                lax.fori_loop(0, hcpt, body, 0)
                for u in (hcpt - 2, hcpt - 1):
                    pltpu.make_async_copy(
                        rows.at[u % 2], acc_sp.at[didx.at[u]], ssem).wait()

        @pl.when(c == 0)
        def _():
            run(ga_hbm)
        @pl.when(c == 1)
        def _():
            run(gb_hbm)

        plsc.subcore_barrier()

        def drain(out_hbm):
            sl = pl.ds(r0, rpt)
            pltpu.sync_copy(acc_sp.at[sl], out_hbm.at[sl])
            @pl.when(s == NS - 1)
            def _():
                tl = pl.ds(NS * rpt, tail)
                pltpu.sync_copy(acc_sp.at[tl], out_hbm.at[tl])

        @pl.when(c == 0)
        def _():
            drain(acca_hbm)
        @pl.when(c == 1)
        def _():
            drain(accb_hbm)

    return k


_RB = 1000


def _tc_scale_matmul_body(x_ref, w_ref, cnt_ref, ga_ref, gb_ref, dinv_ref):
    xb = x_ref[...]
    hh = jnp.dot(xb, w_ref[...], preferred_element_type=F32)
    dinv = lax.rsqrt(cnt_ref[...] + 1.0)
    g = hh * dinv
    half = g.shape[1] // 2
    ga_ref[...] = g[:, :half]
    gb_ref[...] = g[:, half:]
    dinv_ref[...] = dinv


def _tc1_call(x, w1, cnt):
    n, d = x.shape
    h = d // 2
    grid = (n // _RB,)
    return pl.pallas_call(
        _tc_scale_matmul_body,
        grid=grid,
        in_specs=[
            pl.BlockSpec((_RB, d), lambda i: (i, 0)),
            pl.BlockSpec((d, d), lambda i: (0, 0)),
            pl.BlockSpec((_RB, 1), lambda i: (i, 0)),
        ],
        out_specs=[
            pl.BlockSpec((_RB, h), lambda i: (i, 0)),
            pl.BlockSpec((_RB, h), lambda i: (i, 0)),
            pl.BlockSpec((_RB, 1), lambda i: (i, 0)),
        ],
        out_shape=[
            jax.ShapeDtypeStruct((n, h), F32),
            jax.ShapeDtypeStruct((n, h), F32),
            jax.ShapeDtypeStruct((n, 1), F32),
        ],
    )(x, w1, cnt)


def _post_conv(acc_a, acc_b, g_a, g_b, xres, dinv, b, gamma, beta):
    acc = jnp.concatenate([acc_a, acc_b], axis=1)
    g = jnp.concatenate([g_a, g_b], axis=1)
    conv = dinv * (acc + g) + b
    z = jnp.maximum(conv, 0.0) + xres
    mu = jnp.mean(z, axis=1, keepdims=True)
    var = jnp.mean((z - mu) ** 2, axis=1, keepdims=True)
    return (z - mu) * lax.rsqrt(var + 1e-5) * gamma + beta


def _tc_mid_body(acca_ref, accb_ref, ga_ref, gb_ref, x_ref, dinv_ref,
                 b_ref, gm_ref, bt_ref, w2_ref,
                 x2_ref, g2a_ref, g2b_ref):
    dinv = dinv_ref[...]
    xn = _post_conv(acca_ref[...], accb_ref[...], ga_ref[...], gb_ref[...],
                    x_ref[...], dinv, b_ref[...], gm_ref[...], bt_ref[...])
    x2_ref[...] = xn
    h2 = jnp.dot(xn, w2_ref[...], preferred_element_type=F32)
    g2 = h2 * dinv
    half = g2.shape[1] // 2
    g2a_ref[...] = g2[:, :half]
    g2b_ref[...] = g2[:, half:]


def _tc2_call(acc_a, acc_b, g_a, g_b, x, dinv, b1, gamma, beta, w2):
    n, h = acc_a.shape
    d = 2 * h
    grid = (n // _RB,)
    bs_h = pl.BlockSpec((_RB, h), lambda i: (i, 0))
    bs_d = pl.BlockSpec((_RB, d), lambda i: (i, 0))
    bs_1 = pl.BlockSpec((_RB, 1), lambda i: (i, 0))
    bs_v = pl.BlockSpec((1, d), lambda i: (0, 0))
    return pl.pallas_call(
        _tc_mid_body,
        grid=grid,
        in_specs=[bs_h, bs_h, bs_h, bs_h, bs_d, bs_1, bs_v, bs_v, bs_v,
                  pl.BlockSpec((d, d), lambda i: (0, 0))],
        out_specs=[bs_d, bs_h, bs_h],
        out_shape=[
            jax.ShapeDtypeStruct((n, d), F32),
            jax.ShapeDtypeStruct((n, h), F32),
            jax.ShapeDtypeStruct((n, h), F32),
        ],
    )(acc_a, acc_b, g_a, g_b, x, dinv, b1, gamma, beta, w2)


def _tc_final_body(acca_ref, accb_ref, ga_ref, gb_ref, x_ref, dinv_ref,
                   b_ref, gm_ref, bt_ref, out_ref):
    out_ref[...] = _post_conv(acca_ref[...], accb_ref[...], ga_ref[...],
                              gb_ref[...], x_ref[...], dinv_ref[...],
                              b_ref[...], gm_ref[...], bt_ref[...])


def _tc3_call(acc_a, acc_b, g_a, g_b, x, dinv, b2, gamma, beta):
    n, h = acc_a.shape
    d = 2 * h
    grid = (n // _RB,)
    bs_h = pl.BlockSpec((_RB, h), lambda i: (i, 0))
    bs_d = pl.BlockSpec((_RB, d), lambda i: (i, 0))
    bs_1 = pl.BlockSpec((_RB, 1), lambda i: (i, 0))
    bs_v = pl.BlockSpec((1, d), lambda i: (0, 0))
    return pl.pallas_call(
        _tc_final_body,
        grid=grid,
        in_specs=[bs_h, bs_h, bs_h, bs_h, bs_d, bs_1, bs_v, bs_v, bs_v],
        out_specs=bs_d,
        out_shape=jax.ShapeDtypeStruct((n, d), F32),
    )(acc_a, acc_b, g_a, g_b, x, dinv, b2, gamma, beta)


def kernel(entity_table, W1, b1, W2, b2, gamma, beta, entity_ids, edge_index):
    num_ent, d = entity_table.shape
    n = entity_ids.shape[0]
    e = edge_index.shape[1]
    h = d // 2

    n_chunks = _ceil_to(-(-e // CW), NC * NS * G)
    epad = n_chunks * CW
    src = edge_index[0].astype(I32)
    dst = edge_index[1].astype(I32)
    src2d = jnp.concatenate(
        [src, jnp.zeros((epad - e,), I32)]).reshape(n_chunks, CW)
    dst2d = jnp.concatenate(
        [dst, jnp.full((epad - e,), n, I32)]).reshape(n_chunks, CW)
    ids = entity_ids.astype(I32)

    zeros640 = jnp.zeros((640,), F32)
    ones128 = jnp.ones((CW,), F32)
    rpt = (n // NS) & ~7
    zrows = jnp.zeros((rpt, h), F32)

    x, cnt0, cnt1 = _build_gather_deg(num_ent, n, d, n_chunks)(
        entity_table, ids, dst2d, zeros640, ones128)
    cnt = (cnt0[:n] + cnt1[:n]).reshape(n, 1)

    b1r = b1.reshape(1, d)
    b2r = b2.reshape(1, d)
    gmr = gamma.reshape(1, d)
    btr = beta.reshape(1, d)

    g1a, g1b, dinv = _tc1_call(x, W1, cnt)

    msg = _build_message(n, h, n_chunks)
    acc1a, acc1b = msg(g1a, g1b, src2d, dst2d, zrows)

    x2, g2a, g2b = _tc2_call(acc1a, acc1b, g1a, g1b, x, dinv,
                             b1r, gmr, btr, W2)

    acc2a, acc2b = msg(g2a, g2b, src2d, dst2d, zrows)

    out = _tc3_call(acc2a, acc2b, g2a, g2b, x2, dinv, b2r, gmr, btr)
    return out


# speedup vs baseline: 1.8785x; 1.8785x over previous
import functools

import jax
import jax.numpy as jnp
from jax import lax
from jax.experimental import pallas as pl
from jax.experimental.pallas import tpu as pltpu
from jax.experimental.pallas import tpu_sc as plsc

F32 = jnp.float32
I32 = jnp.int32

NC = 2
NS = 16
CW = 128
G = 4


def _ceil_to(x, m):
    return ((x + m - 1) // m) * m


@functools.lru_cache(maxsize=None)
def _build_gather_deg(num_ent, n, d, n_chunks):
    rows_chunk = 80
    x_chunks = -(-n // rows_chunk)
    xg_iters = -(-x_chunks // (NC * NS))
    npad = _ceil_to(n + 1, 640)
    cpt = n_chunks // (NC * NS)

    mesh = plsc.VectorSubcoreMesh(core_axis_name="c", subcore_axis_name="s",
                                  num_cores=NC, num_subcores=NS)

    @functools.partial(
        pl.kernel,
        mesh=mesh,
        out_type=(
            jax.ShapeDtypeStruct((n, d), F32),
            jax.ShapeDtypeStruct((npad,), F32),
            jax.ShapeDtypeStruct((npad,), F32),
        ),
        scratch_types=[
            pltpu.VMEM((rows_chunk,), I32),
            pltpu.VMEM((rows_chunk, d), F32),
            pltpu.VMEM((cpt, CW), I32),
            pltpu.VMEM((CW,), F32),
            pltpu.VMEM_SHARED((npad,), F32),
            pltpu.SemaphoreType.DMA,
            pltpu.SemaphoreType.DMA,
        ],
    )
    def k(tab_hbm, ids_hbm, dst2d_hbm, zeros_hbm, ones_hbm,
          x_hbm, cnt0_hbm, cnt1_hbm,
          idbuf, rowbuf, dstbuf, onesbuf, cnt_sp, sem, ssem):
        c = lax.axis_index("c")
        s = lax.axis_index("s")
        w = s * NC + c

        pltpu.sync_copy(zeros_hbm, cnt_sp.at[pl.ds(s * 640, 640)])
        pltpu.sync_copy(ones_hbm, onesbuf)
        base_chunk = (c * NS + s) * cpt
        pltpu.sync_copy(dst2d_hbm.at[pl.ds(base_chunk, cpt)], dstbuf)
        plsc.subcore_barrier()

        def dg(j, carry):
            pltpu.async_copy(onesbuf, cnt_sp.at[dstbuf.at[j]], ssem, add=True)
            return carry
        lax.fori_loop(0, cpt, dg, 0)

        def xg(j, carry):
            cid = w + NC * NS * j
            @pl.when(cid < x_chunks)
            def _():
                base = cid * rows_chunk
                pltpu.sync_copy(ids_hbm.at[pl.ds(base, rows_chunk)], idbuf)
                pltpu.async_copy(tab_hbm.at[idbuf], rowbuf, sem).wait()
                pltpu.sync_copy(rowbuf, x_hbm.at[pl.ds(base, rows_chunk)])
            return carry
        lax.fori_loop(0, xg_iters, xg, 0)

        def dw(j, carry):
            pltpu.make_async_copy(onesbuf, cnt_sp.at[dstbuf.at[j]], ssem).wait()
            return carry
        lax.fori_loop(0, cpt, dw, 0)
        plsc.subcore_barrier()
        @pl.when(c == 0)
        def _():
            pltpu.sync_copy(cnt_sp.at[pl.ds(s * 640, 640)],
                            cnt0_hbm.at[pl.ds(s * 640, 640)])
        @pl.when(c == 1)
        def _():
            pltpu.sync_copy(cnt_sp.at[pl.ds(s * 640, 640)],
                            cnt1_hbm.at[pl.ds(s * 640, 640)])

    return k


@functools.lru_cache(maxsize=None)
def _build_message(n, h, n_chunks):
    assert n_chunks % (NS * 2) == 0
    cpt = n_chunks // NS
    hcpt = cpt // 2
    npad = _ceil_to(n + 16, 8)
    rpt = (n // NS) & ~7
    tail = n - NS * rpt
    zpad = npad - n + tail

    mesh = plsc.VectorSubcoreMesh(core_axis_name="c", subcore_axis_name="s",
                                  num_cores=NC, num_subcores=NS)

    @functools.partial(
        pl.kernel,
        mesh=mesh,
        out_type=(
            jax.ShapeDtypeStruct((n, h), F32),
            jax.ShapeDtypeStruct((n, h), F32),
        ),
        scratch_types=[
            pltpu.VMEM((hcpt, CW), I32),
            pltpu.VMEM((hcpt, CW), I32),
            pltpu.VMEM((2, CW, h), F32),
            pltpu.VMEM_SHARED((npad, h), F32),
            pltpu.SemaphoreType.DMA,
            pltpu.SemaphoreType.DMA,
        ],
    )
    def k(ga_hbm, gb_hbm, src2d_hbm, dst2d_hbm, zrows_hbm,
          acca_hbm, accb_hbm,
          sidx, didx, rows, acc_sp, gsem, ssem):
        c = lax.axis_index("c")
        s = lax.axis_index("s")
        r0 = s * rpt

        pltpu.sync_copy(zrows_hbm.at[pl.ds(0, rpt)], acc_sp.at[pl.ds(r0, rpt)])
        @pl.when(s == NS - 1)
        def _():
            pltpu.sync_copy(zrows_hbm.at[pl.ds(0, zpad)],
                            acc_sp.at[pl.ds(NS * rpt, zpad)])
        plsc.subcore_barrier()

        def run(g_hbm):
            for half in range(2):
                base_chunk = s * cpt + half * hcpt
                pltpu.sync_copy(src2d_hbm.at[pl.ds(base_chunk, hcpt)], sidx)
                pltpu.sync_copy(dst2d_hbm.at[pl.ds(base_chunk, hcpt)], didx)

                def body(j, carry):
                    b = lax.rem(j, 2)
                    @pl.when(j >= 2)
                    def _():
                        pltpu.make_async_copy(
                            rows.at[b], acc_sp.at[didx.at[j - 2]],
                            ssem).wait()
                    pltpu.sync_copy(g_hbm.at[sidx.at[j]], rows.at[b])
                    pltpu.async_copy(rows.at[b], acc_sp.at[didx.at[j]],
                                     ssem, add=True)
                    return carry
                lax.fori_loop(0, hcpt, body, 0)
                for u in (hcpt - 2, hcpt - 1):
                    pltpu.make_async_copy(
                        rows.at[u % 2], acc_sp.at[didx.at[u]], ssem).wait()

        @pl.when(c == 0)
        def _():
            run(ga_hbm)
        @pl.when(c == 1)
        def _():
            run(gb_hbm)

        plsc.subcore_barrier()

        def drain(out_hbm):
            sl = pl.ds(r0, rpt)
            pltpu.sync_copy(acc_sp.at[sl], out_hbm.at[sl])
            @pl.when(s == NS - 1)
            def _():
                tl = pl.ds(NS * rpt, tail)
                pltpu.sync_copy(acc_sp.at[tl], out_hbm.at[tl])

        @pl.when(c == 0)
        def _():
            drain(acca_hbm)
        @pl.when(c == 1)
        def _():
            drain(accb_hbm)

    return k


_RB = 1000


def _tc_scale_matmul_body(x_ref, w_ref, cnt_ref, ga_ref, gb_ref, dinv_ref):
    xb = x_ref[...]
    hh = jnp.dot(xb, w_ref[...], preferred_element_type=F32)
    dinv = lax.rsqrt(cnt_ref[...] + 1.0)
    g = hh * dinv
    half = g.shape[1] // 2
    ga_ref[...] = g[:, :half]
    gb_ref[...] = g[:, half:]
    dinv_ref[...] = dinv


def _tc1_call(x, w1, cnt):
    n, d = x.shape
    h = d // 2
    grid = (n // _RB,)
    return pl.pallas_call(
        _tc_scale_matmul_body,
        grid=grid,
        in_specs=[
            pl.BlockSpec((_RB, d), lambda i: (i, 0)),
            pl.BlockSpec((d, d), lambda i: (0, 0)),
            pl.BlockSpec((_RB, 1), lambda i: (i, 0)),
        ],
        out_specs=[
            pl.BlockSpec((_RB, h), lambda i: (i, 0)),
            pl.BlockSpec((_RB, h), lambda i: (i, 0)),
            pl.BlockSpec((_RB, 1), lambda i: (i, 0)),
        ],
        out_shape=[
            jax.ShapeDtypeStruct((n, h), F32),
            jax.ShapeDtypeStruct((n, h), F32),
            jax.ShapeDtypeStruct((n, 1), F32),
        ],
    )(x, w1, cnt)


def _post_conv(acc_a, acc_b, g_a, g_b, xres, dinv, b, gamma, beta):
    acc = jnp.concatenate([acc_a, acc_b], axis=1)
    g = jnp.concatenate([g_a, g_b], axis=1)
    conv = dinv * (acc + g) + b
    z = jnp.maximum(conv, 0.0) + xres
    mu = jnp.mean(z, axis=1, keepdims=True)
    var = jnp.mean((z - mu) ** 2, axis=1, keepdims=True)
    return (z - mu) * lax.rsqrt(var + 1e-5) * gamma + beta


def _tc_mid_body(acca_ref, accb_ref, ga_ref, gb_ref, x_ref, dinv_ref,
                 b_ref, gm_ref, bt_ref, w2_ref,
                 x2_ref, g2a_ref, g2b_ref):
    dinv = dinv_ref[...]
    xn = _post_conv(acca_ref[...], accb_ref[...], ga_ref[...], gb_ref[...],
                    x_ref[...], dinv, b_ref[...], gm_ref[...], bt_ref[...])
    x2_ref[...] = xn
    h2 = jnp.dot(xn, w2_ref[...], preferred_element_type=F32)
    g2 = h2 * dinv
    half = g2.shape[1] // 2
    g2a_ref[...] = g2[:, :half]
    g2b_ref[...] = g2[:, half:]


def _tc2_call(acc_a, acc_b, g_a, g_b, x, dinv, b1, gamma, beta, w2):
    n, h = acc_a.shape
    d = 2 * h
    grid = (n // _RB,)
    bs_h = pl.BlockSpec((_RB, h), lambda i: (i, 0))
    bs_d = pl.BlockSpec((_RB, d), lambda i: (i, 0))
    bs_1 = pl.BlockSpec((_RB, 1), lambda i: (i, 0))
    bs_v = pl.BlockSpec((1, d), lambda i: (0, 0))
    return pl.pallas_call(
        _tc_mid_body,
        grid=grid,
        in_specs=[bs_h, bs_h, bs_h, bs_h, bs_d, bs_1, bs_v, bs_v, bs_v,
                  pl.BlockSpec((d, d), lambda i: (0, 0))],
        out_specs=[bs_d, bs_h, bs_h],
        out_shape=[
            jax.ShapeDtypeStruct((n, d), F32),
            jax.ShapeDtypeStruct((n, h), F32),
            jax.ShapeDtypeStruct((n, h), F32),
        ],
    )(acc_a, acc_b, g_a, g_b, x, dinv, b1, gamma, beta, w2)


def _tc_final_body(acca_ref, accb_ref, ga_ref, gb_ref, x_ref, dinv_ref,
                   b_ref, gm_ref, bt_ref, out_ref):
    out_ref[...] = _post_conv(acca_ref[...], accb_ref[...], ga_ref[...],
                              gb_ref[...], x_ref[...], dinv_ref[...],
                              b_ref[...], gm_ref[...], bt_ref[...])


def _tc3_call(acc_a, acc_b, g_a, g_b, x, dinv, b2, gamma, beta):
    n, h = acc_a.shape
    d = 2 * h
    grid = (n // _RB,)
    bs_h = pl.BlockSpec((_RB, h), lambda i: (i, 0))
    bs_d = pl.BlockSpec((_RB, d), lambda i: (i, 0))
    bs_1 = pl.BlockSpec((_RB, 1), lambda i: (i, 0))
    bs_v = pl.BlockSpec((1, d), lambda i: (0, 0))
    return pl.pallas_call(
        _tc_final_body,
        grid=grid,
        in_specs=[bs_h, bs_h, bs_h, bs_h, bs_d, bs_1, bs_v, bs_v, bs_v],
        out_specs=bs_d,
        out_shape=jax.ShapeDtypeStruct((n, d), F32),
    )(acc_a, acc_b, g_a, g_b, x, dinv, b2, gamma, beta)


def kernel(entity_table, W1, b1, W2, b2, gamma, beta, entity_ids, edge_index):
    num_ent, d = entity_table.shape
    n = entity_ids.shape[0]
    e = edge_index.shape[1]
    h = d // 2

    n_chunks = _ceil_to(-(-e // CW), NC * NS * G)
    epad = n_chunks * CW
    src = edge_index[0].astype(I32)
    dst = edge_index[1].astype(I32)
    pad_i = jnp.arange(epad - e, dtype=I32) % 16
    src2d = jnp.concatenate([src, pad_i]).reshape(n_chunks, CW)
    dst2d = jnp.concatenate([dst, n + pad_i]).reshape(n_chunks, CW)
    ids = entity_ids.astype(I32)

    zeros640 = jnp.zeros((640,), F32)
    ones128 = jnp.ones((CW,), F32)
    rpt = (n // NS) & ~7
    zrows = jnp.zeros((rpt, h), F32)

    x, cnt0, cnt1 = _build_gather_deg(num_ent, n, d, n_chunks)(
        entity_table, ids, dst2d, zeros640, ones128)
    cnt = (cnt0[:n] + cnt1[:n]).reshape(n, 1)

    b1r = b1.reshape(1, d)
    b2r = b2.reshape(1, d)
    gmr = gamma.reshape(1, d)
    btr = beta.reshape(1, d)

    g1a, g1b, dinv = _tc1_call(x, W1, cnt)

    msg = _build_message(n, h, n_chunks)
    acc1a, acc1b = msg(g1a, g1b, src2d, dst2d, zrows)

    x2, g2a, g2b = _tc2_call(acc1a, acc1b, g1a, g1b, x, dinv,
                             b1r, gmr, btr, W2)

    acc2a, acc2b = msg(g2a, g2b, src2d, dst2d, zrows)

    out = _tc3_call(acc2a, acc2b, g2a, g2b, x2, dinv, b2r, gmr, btr)
    return out


# speedup vs baseline: 2.1099x; 1.1232x over previous
import functools

import jax
import jax.numpy as jnp
from jax import lax
from jax.experimental import pallas as pl
from jax.experimental.pallas import tpu as pltpu
from jax.experimental.pallas import tpu_sc as plsc

F32 = jnp.float32
I32 = jnp.int32

NC = 2
NS = 16
CW = 128
G = 4


def _ceil_to(x, m):
    return ((x + m - 1) // m) * m


@functools.lru_cache(maxsize=None)
def _build_gather_deg(num_ent, n, d, n_chunks):
    rows_chunk = 80
    x_chunks = -(-n // rows_chunk)
    xg_iters = -(-x_chunks // (NC * NS))
    npad = _ceil_to(n + 1, 640)
    cpt = n_chunks // (NC * NS)

    mesh = plsc.VectorSubcoreMesh(core_axis_name="c", subcore_axis_name="s",
                                  num_cores=NC, num_subcores=NS)

    @functools.partial(
        pl.kernel,
        mesh=mesh,
        out_type=(
            jax.ShapeDtypeStruct((n, d), F32),
            jax.ShapeDtypeStruct((npad,), F32),
            jax.ShapeDtypeStruct((npad,), F32),
        ),
        scratch_types=[
            pltpu.VMEM((rows_chunk,), I32),
            pltpu.VMEM((rows_chunk, d), F32),
            pltpu.VMEM((cpt, CW), I32),
            pltpu.VMEM((CW,), F32),
            pltpu.VMEM_SHARED((npad,), F32),
            pltpu.SemaphoreType.DMA,
            pltpu.SemaphoreType.DMA,
        ],
    )
    def k(tab_hbm, ids_hbm, dst2d_hbm, zeros_hbm, ones_hbm,
          x_hbm, cnt0_hbm, cnt1_hbm,
          idbuf, rowbuf, dstbuf, onesbuf, cnt_sp, sem, ssem):
        c = lax.axis_index("c")
        s = lax.axis_index("s")
        w = s * NC + c

        pltpu.sync_copy(zeros_hbm, cnt_sp.at[pl.ds(s * 640, 640)])
        pltpu.sync_copy(ones_hbm, onesbuf)
        base_chunk = (c * NS + s) * cpt
        pltpu.sync_copy(dst2d_hbm.at[pl.ds(base_chunk, cpt)], dstbuf)
        plsc.subcore_barrier()

        def dg(j, carry):
            pltpu.async_copy(onesbuf, cnt_sp.at[dstbuf.at[j]], ssem, add=True)
            return carry
        lax.fori_loop(0, cpt, dg, 0)

        def xg(j, carry):
            cid = w + NC * NS * j
            @pl.when(cid < x_chunks)
            def _():
                base = cid * rows_chunk
                pltpu.sync_copy(ids_hbm.at[pl.ds(base, rows_chunk)], idbuf)
                pltpu.async_copy(tab_hbm.at[idbuf], rowbuf, sem).wait()
                pltpu.sync_copy(rowbuf, x_hbm.at[pl.ds(base, rows_chunk)])
            return carry
        lax.fori_loop(0, xg_iters, xg, 0)

        def dw(j, carry):
            pltpu.make_async_copy(onesbuf, cnt_sp.at[dstbuf.at[j]], ssem).wait()
            return carry
        lax.fori_loop(0, cpt, dw, 0)
        plsc.subcore_barrier()
        @pl.when(c == 0)
        def _():
            pltpu.sync_copy(cnt_sp.at[pl.ds(s * 640, 640)],
                            cnt0_hbm.at[pl.ds(s * 640, 640)])
        @pl.when(c == 1)
        def _():
            pltpu.sync_copy(cnt_sp.at[pl.ds(s * 640, 640)],
                            cnt1_hbm.at[pl.ds(s * 640, 640)])

    return k


@functools.lru_cache(maxsize=None)
def _build_message(n, h, n_chunks):
    assert n_chunks % (NS * 2) == 0
    cpt = n_chunks // NS
    hcpt = cpt // 2
    npad = _ceil_to(n + 16, 8)
    rpt = (n // NS) & ~7
    tail = n - NS * rpt
    zpad = npad - n + tail

    mesh = plsc.VectorSubcoreMesh(core_axis_name="c", subcore_axis_name="s",
                                  num_cores=NC, num_subcores=NS)

    @functools.partial(
        pl.kernel,
        mesh=mesh,
        out_type=(
            jax.ShapeDtypeStruct((n, h), F32),
            jax.ShapeDtypeStruct((n, h), F32),
        ),
        scratch_types=[
            pltpu.VMEM((hcpt, CW), I32),
            pltpu.VMEM((hcpt, CW), I32),
            pltpu.VMEM((2, CW, h), F32),
            pltpu.VMEM_SHARED((npad, h), F32),
            pltpu.SemaphoreType.DMA((2,)),
            pltpu.SemaphoreType.DMA((2,)),
        ],
    )
    def k(ga_hbm, gb_hbm, src2d_hbm, dst2d_hbm, zrows_hbm,
          acca_hbm, accb_hbm,
          sidx, didx, rows, acc_sp, gsem, ssem):
        c = lax.axis_index("c")
        s = lax.axis_index("s")
        r0 = s * rpt

        pltpu.sync_copy(zrows_hbm.at[pl.ds(0, rpt)], acc_sp.at[pl.ds(r0, rpt)])
        @pl.when(s == NS - 1)
        def _():
            pltpu.sync_copy(zrows_hbm.at[pl.ds(0, zpad)],
                            acc_sp.at[pl.ds(NS * rpt, zpad)])
        plsc.subcore_barrier()

        def run(g_hbm):
            def wait_gather(b):
                pltpu.make_async_copy(g_hbm.at[pl.ds(0, CW)], rows.at[0],
                                      gsem.at[b]).wait()

            def wait_scatter(b):
                pltpu.make_async_copy(rows.at[0], acc_sp.at[pl.ds(0, CW)],
                                      ssem.at[b]).wait()

            for half in range(2):
                base_chunk = s * cpt + half * hcpt
                pltpu.sync_copy(src2d_hbm.at[pl.ds(base_chunk, hcpt)], sidx)
                pltpu.sync_copy(dst2d_hbm.at[pl.ds(base_chunk, hcpt)], didx)

                pltpu.async_copy(g_hbm.at[sidx.at[0]], rows.at[0],
                                 gsem.at[0])

                def body(j, carry):
                    b = lax.rem(j, 2)
                    nb = 1 - b
                    @pl.when(j >= 1)
                    def _():
                        wait_scatter(nb)
                    @pl.when(j + 1 < hcpt)
                    def _():
                        pltpu.async_copy(g_hbm.at[sidx.at[j + 1]],
                                         rows.at[nb], gsem.at[nb])
                    wait_gather(b)
                    pltpu.async_copy(rows.at[b], acc_sp.at[didx.at[j]],
                                     ssem.at[b], add=True)
                    return carry
                lax.fori_loop(0, hcpt, body, 0)
                wait_scatter((hcpt - 1) % 2)

        @pl.when(c == 0)
        def _():
            run(ga_hbm)
        @pl.when(c == 1)
        def _():
            run(gb_hbm)

        plsc.subcore_barrier()

        def drain(out_hbm):
            sl = pl.ds(r0, rpt)
            pltpu.sync_copy(acc_sp.at[sl], out_hbm.at[sl])
            @pl.when(s == NS - 1)
            def _():
                tl = pl.ds(NS * rpt, tail)
                pltpu.sync_copy(acc_sp.at[tl], out_hbm.at[tl])

        @pl.when(c == 0)
        def _():
            drain(acca_hbm)
        @pl.when(c == 1)
        def _():
            drain(accb_hbm)

    return k


_RB = 1000


def _tc_scale_matmul_body(x_ref, w_ref, cnt_ref, ga_ref, gb_ref, dinv_ref):
    xb = x_ref[...]
    hh = jnp.dot(xb, w_ref[...], preferred_element_type=F32)
    dinv = lax.rsqrt(cnt_ref[...] + 1.0)
    g = hh * dinv
    half = g.shape[1] // 2
    ga_ref[...] = g[:, :half]
    gb_ref[...] = g[:, half:]
    dinv_ref[...] = dinv


def _tc1_call(x, w1, cnt):
    n, d = x.shape
    h = d // 2
    grid = (n // _RB,)
    return pl.pallas_call(
        _tc_scale_matmul_body,
        grid=grid,
        in_specs=[
            pl.BlockSpec((_RB, d), lambda i: (i, 0)),
            pl.BlockSpec((d, d), lambda i: (0, 0)),
            pl.BlockSpec((_RB, 1), lambda i: (i, 0)),
        ],
        out_specs=[
            pl.BlockSpec((_RB, h), lambda i: (i, 0)),
            pl.BlockSpec((_RB, h), lambda i: (i, 0)),
            pl.BlockSpec((_RB, 1), lambda i: (i, 0)),
        ],
        out_shape=[
            jax.ShapeDtypeStruct((n, h), F32),
            jax.ShapeDtypeStruct((n, h), F32),
            jax.ShapeDtypeStruct((n, 1), F32),
        ],
    )(x, w1, cnt)


def _post_conv(acc_a, acc_b, g_a, g_b, xres, dinv, b, gamma, beta):
    acc = jnp.concatenate([acc_a, acc_b], axis=1)
    g = jnp.concatenate([g_a, g_b], axis=1)
    conv = dinv * (acc + g) + b
    z = jnp.maximum(conv, 0.0) + xres
    mu = jnp.mean(z, axis=1, keepdims=True)
    var = jnp.mean((z - mu) ** 2, axis=1, keepdims=True)
    return (z - mu) * lax.rsqrt(var + 1e-5) * gamma + beta


def _tc_mid_body(acca_ref, accb_ref, ga_ref, gb_ref, x_ref, dinv_ref,
                 b_ref, gm_ref, bt_ref, w2_ref,
                 x2_ref, g2a_ref, g2b_ref):
    dinv = dinv_ref[...]
    xn = _post_conv(acca_ref[...], accb_ref[...], ga_ref[...], gb_ref[...],
                    x_ref[...], dinv, b_ref[...], gm_ref[...], bt_ref[...])
    x2_ref[...] = xn
    h2 = jnp.dot(xn, w2_ref[...], preferred_element_type=F32)
    g2 = h2 * dinv
    half = g2.shape[1] // 2
    g2a_ref[...] = g2[:, :half]
    g2b_ref[...] = g2[:, half:]


def _tc2_call(acc_a, acc_b, g_a, g_b, x, dinv, b1, gamma, beta, w2):
    n, h = acc_a.shape
    d = 2 * h
    grid = (n // _RB,)
    bs_h = pl.BlockSpec((_RB, h), lambda i: (i, 0))
    bs_d = pl.BlockSpec((_RB, d), lambda i: (i, 0))
    bs_1 = pl.BlockSpec((_RB, 1), lambda i: (i, 0))
    bs_v = pl.BlockSpec((1, d), lambda i: (0, 0))
    return pl.pallas_call(
        _tc_mid_body,
        grid=grid,
        in_specs=[bs_h, bs_h, bs_h, bs_h, bs_d, bs_1, bs_v, bs_v, bs_v,
                  pl.BlockSpec((d, d), lambda i: (0, 0))],
        out_specs=[bs_d, bs_h, bs_h],
        out_shape=[
            jax.ShapeDtypeStruct((n, d), F32),
            jax.ShapeDtypeStruct((n, h), F32),
            jax.ShapeDtypeStruct((n, h), F32),
        ],
    )(acc_a, acc_b, g_a, g_b, x, dinv, b1, gamma, beta, w2)


def _tc_final_body(acca_ref, accb_ref, ga_ref, gb_ref, x_ref, dinv_ref,
                   b_ref, gm_ref, bt_ref, out_ref):
    out_ref[...] = _post_conv(acca_ref[...], accb_ref[...], ga_ref[...],
                              gb_ref[...], x_ref[...], dinv_ref[...],
                              b_ref[...], gm_ref[...], bt_ref[...])


def _tc3_call(acc_a, acc_b, g_a, g_b, x, dinv, b2, gamma, beta):
    n, h = acc_a.shape
    d = 2 * h
    grid = (n // _RB,)
    bs_h = pl.BlockSpec((_RB, h), lambda i: (i, 0))
    bs_d = pl.BlockSpec((_RB, d), lambda i: (i, 0))
    bs_1 = pl.BlockSpec((_RB, 1), lambda i: (i, 0))
    bs_v = pl.BlockSpec((1, d), lambda i: (0, 0))
    return pl.pallas_call(
        _tc_final_body,
        grid=grid,
        in_specs=[bs_h, bs_h, bs_h, bs_h, bs_d, bs_1, bs_v, bs_v, bs_v],
        out_specs=bs_d,
        out_shape=jax.ShapeDtypeStruct((n, d), F32),
    )(acc_a, acc_b, g_a, g_b, x, dinv, b2, gamma, beta)


def kernel(entity_table, W1, b1, W2, b2, gamma, beta, entity_ids, edge_index):
    num_ent, d = entity_table.shape
    n = entity_ids.shape[0]
    e = edge_index.shape[1]
    h = d // 2

    n_chunks = _ceil_to(-(-e // CW), NC * NS * G)
    epad = n_chunks * CW
    src = edge_index[0].astype(I32)
    dst = edge_index[1].astype(I32)
    pad_i = jnp.arange(epad - e, dtype=I32) % 16
    src2d = jnp.concatenate([src, pad_i]).reshape(n_chunks, CW)
    dst2d = jnp.concatenate([dst, n + pad_i]).reshape(n_chunks, CW)
    ids = entity_ids.astype(I32)

    zeros640 = jnp.zeros((640,), F32)
    ones128 = jnp.ones((CW,), F32)
    rpt = (n // NS) & ~7
    zrows = jnp.zeros((rpt, h), F32)

    x, cnt0, cnt1 = _build_gather_deg(num_ent, n, d, n_chunks)(
        entity_table, ids, dst2d, zeros640, ones128)
    cnt = (cnt0[:n] + cnt1[:n]).reshape(n, 1)

    b1r = b1.reshape(1, d)
    b2r = b2.reshape(1, d)
    gmr = gamma.reshape(1, d)
    btr = beta.reshape(1, d)

    g1a, g1b, dinv = _tc1_call(x, W1, cnt)

    msg = _build_message(n, h, n_chunks)
    acc1a, acc1b = msg(g1a, g1b, src2d, dst2d, zrows)

    x2, g2a, g2b = _tc2_call(acc1a, acc1b, g1a, g1b, x, dinv,
                             b1r, gmr, btr, W2)

    acc2a, acc2b = msg(g2a, g2b, src2d, dst2d, zrows)

    out = _tc3_call(acc2a, acc2b, g2a, g2b, x2, dinv, b2r, gmr, btr)
    return out
